# Initial kernel scaffold; baseline (speedup 1.0000x reference)
import jax
import jax.numpy as jnp
from jax.experimental import pallas as pl


def kernel(x, edge_index, batch, W_in, b_in, W_conv, b_conv, gamma, beta, W_fc, b_fc):
    raise NotImplementedError("write your pallas kernel here")


# speedup vs baseline: 15.0723x; 15.0723x over previous
import functools

import jax
import jax.numpy as jnp
from jax import lax
from jax.experimental import pallas as pl
from jax.experimental.pallas import tpu as pltpu
from jax.experimental.pallas import tpu_sc as plsc

N = 10000
E = 320000
D = 128
DOUT = 64
G = 16
EPS = 1e-5

NC = 2
NS = 16
NW = NC * NS
EPW = E // NW
CH = 128
ROWS = EPW // CH + (1 if EPW % CH else 0)
ROWS = 80
EPAD = ROWS * CH
NPAD = EPAD
RPT = NPAD // NS
PAD_DST = N

BB = 2000
NB = N // BB

def _deg_body(dst_hbm, out_hbm, dst_v, hist):
    c = lax.axis_index("c")
    s = lax.axis_index("s")
    wid = s * NC + c
    pltpu.sync_copy(dst_hbm.at[wid], dst_v)

    def zrow(j, carry):
        hist[pl.ds(j * 16, 16)] = jnp.zeros((16,), jnp.float32)
        return carry

    lax.fori_loop(0, EPAD // 16, zrow, 0)
    ones = jnp.ones((16,), jnp.float32)

    def erow(j, carry):
        for k in range(CH // 16):
            v = dst_v[j, pl.ds(k * 16, 16)]
            plsc.addupdate_scatter(hist, [v], ones)
        return carry

    lax.fori_loop(0, ROWS, erow, 0)
    pltpu.sync_copy(hist, out_hbm.at[wid])


@functools.lru_cache(maxsize=None)
def _deg_kernel():
    mesh = plsc.VectorSubcoreMesh(core_axis_name="c", subcore_axis_name="s")
    return pl.kernel(
        _deg_body,
        out_type=jax.ShapeDtypeStruct((NW, EPAD), jnp.float32),
        mesh=mesh,
        scratch_types=[
            pltpu.VMEM((ROWS, CH), jnp.int32),
            pltpu.VMEM((EPAD,), jnp.float32),
        ],
        compiler_params=pltpu.CompilerParams(needs_layout_passes=False),
    )


def _msg_body(g_hbm, src_hbm, dst_hbm, out_hbm, src_v, dst_v, rows_v, acc_sh, sem):
    c = lax.axis_index("c")
    s = lax.axis_index("s")
    wid = s * NC + c
    pltpu.sync_copy(src_hbm.at[wid], src_v)
    pltpu.sync_copy(dst_hbm.at[wid], dst_v)

    def zrow(j, carry):
        for k in range(D // 16):
            rows_v[j, pl.ds(k * 16, 16)] = jnp.zeros((16,), jnp.float32)
        return carry

    lax.fori_loop(0, CH, zrow, 0)
    for k in range(RPT // CH):
        pltpu.sync_copy(rows_v, acc_sh.at[pl.ds(s * RPT + k * CH, CH)])
    plsc.subcore_barrier()

    def chunk(j, carry):
        pltpu.async_copy(g_hbm.at[src_v.at[j]], rows_v, sem).wait()
        pltpu.sync_copy(rows_v, acc_sh.at[dst_v.at[j]], add=True)
        return carry

    lax.fori_loop(0, ROWS, chunk, 0)
    plsc.subcore_barrier()
    for k in range(RPT // CH):
        off = s * RPT + k * CH
        pltpu.sync_copy(acc_sh.at[pl.ds(off, CH)], out_hbm.at[c, pl.ds(off, CH)])


@functools.lru_cache(maxsize=None)
def _msg_kernel():
    mesh = plsc.VectorSubcoreMesh(core_axis_name="c", subcore_axis_name="s")
    return pl.kernel(
        _msg_body,
        out_type=jax.ShapeDtypeStruct((NC, NPAD, D), jnp.float32),
        mesh=mesh,
        scratch_types=[
            pltpu.VMEM((ROWS, CH), jnp.int32),
            pltpu.VMEM((ROWS, CH), jnp.int32),
            pltpu.VMEM((CH, D), jnp.float32),
            pltpu.VMEM_SHARED((NPAD, D), jnp.float32),
            pltpu.SemaphoreType.DMA,
        ],
        compiler_params=pltpu.CompilerParams(needs_layout_passes=False),
    )


def _proj_body(x_ref, win_ref, bin_ref, wconv_ref, degt_ref, h_ref, g_ref):
    h = jnp.dot(x_ref[...], win_ref[...], preferred_element_type=jnp.float32)
    h = h + bin_ref[...]
    hw = jnp.dot(h, wconv_ref[...], preferred_element_type=jnp.float32)
    deg = jnp.sum(degt_ref[...], axis=1, keepdims=True) + 1.0
    dinv = lax.rsqrt(deg)
    h_ref[...] = h
    g_ref[...] = hw * dinv


_proj_kernel = pl.pallas_call(
    _proj_body,
    grid=(NB,),
    in_specs=[
        pl.BlockSpec((BB, D), lambda i: (i, 0)),
        pl.BlockSpec((D, D), lambda i: (0, 0)),
        pl.BlockSpec((1, D), lambda i: (0, 0)),
        pl.BlockSpec((D, D), lambda i: (0, 0)),
        pl.BlockSpec((BB, NW), lambda i: (i, 0)),
    ],
    out_specs=[
        pl.BlockSpec((BB, D), lambda i: (i, 0)),
        pl.BlockSpec((BB, D), lambda i: (i, 0)),
    ],
    out_shape=[
        jax.ShapeDtypeStruct((N, D), jnp.float32),
        jax.ShapeDtypeStruct((N, D), jnp.float32),
    ],
)


def _agg_body(g_ref, p_ref, degt_ref, bconv_ref, h2_ref, s1_ref, s2_ref):
    i = pl.program_id(0)
    deg = jnp.sum(degt_ref[...], axis=1, keepdims=True) + 1.0
    dinv = lax.rsqrt(deg)
    p = p_ref[...]
    h2 = dinv * (g_ref[...] + p[0] + p[1]) + bconv_ref[...]
    h2_ref[...] = h2

    @pl.when(i == 0)
    def _():
        s1_ref[...] = jnp.zeros_like(s1_ref)
        s2_ref[...] = jnp.zeros_like(s2_ref)

    s1_ref[...] += jnp.sum(h2, axis=0, keepdims=True)
    s2_ref[...] += jnp.sum(h2 * h2, axis=0, keepdims=True)


_agg_kernel = pl.pallas_call(
    _agg_body,
    grid=(NB,),
    in_specs=[
        pl.BlockSpec((BB, D), lambda i: (i, 0)),
        pl.BlockSpec((NC, BB, D), lambda i: (0, i, 0)),
        pl.BlockSpec((BB, NW), lambda i: (i, 0)),
        pl.BlockSpec((1, D), lambda i: (0, 0)),
    ],
    out_specs=[
        pl.BlockSpec((BB, D), lambda i: (i, 0)),
        pl.BlockSpec((1, D), lambda i: (0, 0)),
        pl.BlockSpec((1, D), lambda i: (0, 0)),
    ],
    out_shape=[
        jax.ShapeDtypeStruct((N, D), jnp.float32),
        jax.ShapeDtypeStruct((1, D), jnp.float32),
        jax.ShapeDtypeStruct((1, D), jnp.float32),
    ],
)


def _fin_body(h2_ref, h_ref, s1_ref, s2_ref, gamma_ref, beta_ref, batch_ref,
              wfc_ref, bfc_ref, out_ref, pool_acc, cnt_acc):
    i = pl.program_id(0)
    mean = s1_ref[...] * (1.0 / N)
    var = s2_ref[...] * (1.0 / N) - mean * mean
    inv = lax.rsqrt(var + EPS)
    hn = (h2_ref[...] - mean) * inv * gamma_ref[...] + beta_ref[...]
    hf = jnp.maximum(hn, 0.0) + h_ref[...]
    b = batch_ref[...][0]
    seg = lax.broadcasted_iota(jnp.int32, (G, BB), 0)
    mask = (seg == b).astype(jnp.float32)

    @pl.when(i == 0)
    def _():
        pool_acc[...] = jnp.zeros_like(pool_acc)
        cnt_acc[...] = jnp.zeros_like(cnt_acc)

    pool_acc[...] += jnp.dot(mask, hf, preferred_element_type=jnp.float32)
    cnt_acc[...] += jnp.sum(mask, axis=1, keepdims=True)

    @pl.when(i == pl.num_programs(0) - 1)
    def _():
        pooled = pool_acc[...] / jnp.maximum(cnt_acc[...], 1.0)
        z = jnp.dot(pooled, wfc_ref[...], preferred_element_type=jnp.float32)
        z = z + bfc_ref[...]
        out_ref[...] = 1.0 / (1.0 + jnp.exp(-z))


_fin_kernel = pl.pallas_call(
    _fin_body,
    grid=(NB,),
    in_specs=[
        pl.BlockSpec((BB, D), lambda i: (i, 0)),
        pl.BlockSpec((BB, D), lambda i: (i, 0)),
        pl.BlockSpec((1, D), lambda i: (0, 0)),
        pl.BlockSpec((1, D), lambda i: (0, 0)),
        pl.BlockSpec((1, D), lambda i: (0, 0)),
        pl.BlockSpec((1, D), lambda i: (0, 0)),
        pl.BlockSpec((1, 1, BB), lambda i: (i, 0, 0)),
        pl.BlockSpec((D, DOUT), lambda i: (0, 0)),
        pl.BlockSpec((1, DOUT), lambda i: (0, 0)),
    ],
    out_specs=pl.BlockSpec((G, DOUT), lambda i: (0, 0)),
    out_shape=jax.ShapeDtypeStruct((G, DOUT), jnp.float32),
    scratch_shapes=[
        pltpu.VMEM((G, D), jnp.float32),
        pltpu.VMEM((G, 1), jnp.float32),
    ],
)


def kernel(x, edge_index, batch, W_in, b_in, W_conv, b_conv, gamma, beta,
           W_fc, b_fc):
    src = edge_index[0].reshape(NW, EPW)
    dst = edge_index[1].reshape(NW, EPW)
    pad = EPAD - EPW
    src_p = jnp.pad(src, ((0, 0), (0, pad))).reshape(NW, ROWS, CH)
    dst_p = jnp.pad(dst, ((0, 0), (0, pad)),
                    constant_values=PAD_DST).reshape(NW, ROWS, CH)

    degp = _deg_kernel()(dst_p)
    degt = degp[:, :N].T

    h, g = _proj_kernel(x, W_in, b_in.reshape(1, D), W_conv, degt)
    parts = _msg_kernel()(g, src_p, dst_p)
    h2, s1, s2 = _agg_kernel(g, parts, degt, b_conv.reshape(1, D))
    out = _fin_kernel(h2, h, s1, s2, gamma.reshape(1, D), beta.reshape(1, D),
                      batch.reshape(NB, 1, BB), W_fc, b_fc.reshape(1, DOUT))
    return out


# speedup vs baseline: 16.9502x; 1.1246x over previous
import functools

import jax
import jax.numpy as jnp
from jax import lax
from jax.experimental import pallas as pl
from jax.experimental.pallas import tpu as pltpu
from jax.experimental.pallas import tpu_sc as plsc

N = 10000
E = 320000
D = 128
DOUT = 64
G = 16
EPS = 1e-5

NC = 2
NS = 16
NW = NC * NS
EPW = E // NW
CH = 64
ROWS = 160
EPAD = ROWS * CH
NPAD = EPAD
RPT = NPAD // NS
PAD_DST = N

BB = 2000
NB = N // BB

def _deg_body(dst_hbm, out_hbm, dst_v, hist):
    c = lax.axis_index("c")
    s = lax.axis_index("s")
    wid = s * NC + c
    pltpu.sync_copy(dst_hbm.at[wid], dst_v)

    def zrow(j, carry):
        hist[pl.ds(j * 16, 16)] = jnp.zeros((16,), jnp.float32)
        return carry

    lax.fori_loop(0, EPAD // 16, zrow, 0)
    ones = jnp.ones((16,), jnp.float32)

    def erow(j, carry):
        for k in range(CH // 16):
            v = dst_v[j, pl.ds(k * 16, 16)]
            plsc.addupdate_scatter(hist, [v], ones)
        return carry

    lax.fori_loop(0, ROWS, erow, 0)
    pltpu.sync_copy(hist, out_hbm.at[wid])


@functools.lru_cache(maxsize=None)
def _deg_kernel():
    mesh = plsc.VectorSubcoreMesh(core_axis_name="c", subcore_axis_name="s")
    return pl.kernel(
        _deg_body,
        out_type=jax.ShapeDtypeStruct((NW, EPAD), jnp.float32),
        mesh=mesh,
        scratch_types=[
            pltpu.VMEM((ROWS, CH), jnp.int32),
            pltpu.VMEM((EPAD,), jnp.float32),
        ],
        compiler_params=pltpu.CompilerParams(needs_layout_passes=False),
    )


NBUF = 3
SB = 40
NSB = ROWS // SB


def _msg_body(g_hbm, src_hbm, dst_hbm, out_hbm, src_v, dst_v, rows_v, acc_sh,
              gsems, ssems):
    c = lax.axis_index("c")
    s = lax.axis_index("s")
    wid = s * NC + c

    def zrow(j, carry):
        for k in range(D // 16):
            rows_v[0, j, pl.ds(k * 16, 16)] = jnp.zeros((16,), jnp.float32)
        return carry

    lax.fori_loop(0, CH, zrow, 0)
    for k in range(RPT // CH):
        pltpu.sync_copy(rows_v.at[0], acc_sh.at[pl.ds(s * RPT + k * CH, CH)])
    plsc.subcore_barrier()

    for sb in range(NSB):
        pltpu.sync_copy(src_hbm.at[wid, pl.ds(sb * SB, SB)], src_v)
        pltpu.sync_copy(dst_hbm.at[wid, pl.ds(sb * SB, SB)], dst_v)
        g_desc = [None] * NBUF
        s_desc = [None] * NBUF
        for j in range(SB + NBUF - 1):
            if j < SB:
                b = j % NBUF
                if s_desc[b] is not None:
                    s_desc[b].wait()
                    s_desc[b] = None
                g_desc[b] = pltpu.async_copy(
                    g_hbm.at[src_v.at[j]], rows_v.at[b], gsems.at[b])
            jj = j - (NBUF - 1)
            if jj >= 0:
                b2 = jj % NBUF
                g_desc[b2].wait()
                s_desc[b2] = pltpu.async_copy(
                    rows_v.at[b2], acc_sh.at[dst_v.at[jj]], ssems.at[b2],
                    add=True)
        for b in range(NBUF):
            if s_desc[b] is not None:
                s_desc[b].wait()

    plsc.subcore_barrier()
    for k in range(RPT // CH):
        off = s * RPT + k * CH
        pltpu.sync_copy(acc_sh.at[pl.ds(off, CH)], out_hbm.at[c, pl.ds(off, CH)])


@functools.lru_cache(maxsize=None)
def _msg_kernel():
    mesh = plsc.VectorSubcoreMesh(core_axis_name="c", subcore_axis_name="s")
    return pl.kernel(
        _msg_body,
        out_type=jax.ShapeDtypeStruct((NC, NPAD, D), jnp.float32),
        mesh=mesh,
        scratch_types=[
            pltpu.VMEM((SB, CH), jnp.int32),
            pltpu.VMEM((SB, CH), jnp.int32),
            pltpu.VMEM((NBUF, CH, D), jnp.float32),
            pltpu.VMEM_SHARED((NPAD, D), jnp.float32),
            pltpu.SemaphoreType.DMA((NBUF,)),
            pltpu.SemaphoreType.DMA((NBUF,)),
        ],
        compiler_params=pltpu.CompilerParams(needs_layout_passes=False),
    )


def _proj_body(x_ref, win_ref, bin_ref, wconv_ref, degt_ref, h_ref, g_ref):
    h = jnp.dot(x_ref[...], win_ref[...], preferred_element_type=jnp.float32)
    h = h + bin_ref[...]
    hw = jnp.dot(h, wconv_ref[...], preferred_element_type=jnp.float32)
    deg = jnp.sum(degt_ref[...], axis=1, keepdims=True) + 1.0
    dinv = lax.rsqrt(deg)
    h_ref[...] = h
    g_ref[...] = hw * dinv


_proj_kernel = pl.pallas_call(
    _proj_body,
    grid=(NB,),
    in_specs=[
        pl.BlockSpec((BB, D), lambda i: (i, 0)),
        pl.BlockSpec((D, D), lambda i: (0, 0)),
        pl.BlockSpec((1, D), lambda i: (0, 0)),
        pl.BlockSpec((D, D), lambda i: (0, 0)),
        pl.BlockSpec((BB, NW), lambda i: (i, 0)),
    ],
    out_specs=[
        pl.BlockSpec((BB, D), lambda i: (i, 0)),
        pl.BlockSpec((BB, D), lambda i: (i, 0)),
    ],
    out_shape=[
        jax.ShapeDtypeStruct((N, D), jnp.float32),
        jax.ShapeDtypeStruct((N, D), jnp.float32),
    ],
)


def _agg_body(g_ref, p_ref, degt_ref, bconv_ref, h2_ref, s1_ref, s2_ref):
    i = pl.program_id(0)
    deg = jnp.sum(degt_ref[...], axis=1, keepdims=True) + 1.0
    dinv = lax.rsqrt(deg)
    p = p_ref[...]
    h2 = dinv * (g_ref[...] + p[0] + p[1]) + bconv_ref[...]
    h2_ref[...] = h2

    @pl.when(i == 0)
    def _():
        s1_ref[...] = jnp.zeros_like(s1_ref)
        s2_ref[...] = jnp.zeros_like(s2_ref)

    s1_ref[...] += jnp.sum(h2, axis=0, keepdims=True)
    s2_ref[...] += jnp.sum(h2 * h2, axis=0, keepdims=True)


_agg_kernel = pl.pallas_call(
    _agg_body,
    grid=(NB,),
    in_specs=[
        pl.BlockSpec((BB, D), lambda i: (i, 0)),
        pl.BlockSpec((NC, BB, D), lambda i: (0, i, 0)),
        pl.BlockSpec((BB, NW), lambda i: (i, 0)),
        pl.BlockSpec((1, D), lambda i: (0, 0)),
    ],
    out_specs=[
        pl.BlockSpec((BB, D), lambda i: (i, 0)),
        pl.BlockSpec((1, D), lambda i: (0, 0)),
        pl.BlockSpec((1, D), lambda i: (0, 0)),
    ],
    out_shape=[
        jax.ShapeDtypeStruct((N, D), jnp.float32),
        jax.ShapeDtypeStruct((1, D), jnp.float32),
        jax.ShapeDtypeStruct((1, D), jnp.float32),
    ],
)


def _fin_body(h2_ref, h_ref, s1_ref, s2_ref, gamma_ref, beta_ref, batch_ref,
              wfc_ref, bfc_ref, out_ref, pool_acc, cnt_acc):
    i = pl.program_id(0)
    mean = s1_ref[...] * (1.0 / N)
    var = s2_ref[...] * (1.0 / N) - mean * mean
    inv = lax.rsqrt(var + EPS)
    hn = (h2_ref[...] - mean) * inv * gamma_ref[...] + beta_ref[...]
    hf = jnp.maximum(hn, 0.0) + h_ref[...]
    b = batch_ref[...][0]
    seg = lax.broadcasted_iota(jnp.int32, (G, BB), 0)
    mask = (seg == b).astype(jnp.float32)

    @pl.when(i == 0)
    def _():
        pool_acc[...] = jnp.zeros_like(pool_acc)
        cnt_acc[...] = jnp.zeros_like(cnt_acc)

    pool_acc[...] += jnp.dot(mask, hf, preferred_element_type=jnp.float32)
    cnt_acc[...] += jnp.sum(mask, axis=1, keepdims=True)

    @pl.when(i == pl.num_programs(0) - 1)
    def _():
        pooled = pool_acc[...] / jnp.maximum(cnt_acc[...], 1.0)
        z = jnp.dot(pooled, wfc_ref[...], preferred_element_type=jnp.float32)
        z = z + bfc_ref[...]
        out_ref[...] = 1.0 / (1.0 + jnp.exp(-z))


_fin_kernel = pl.pallas_call(
    _fin_body,
    grid=(NB,),
    in_specs=[
        pl.BlockSpec((BB, D), lambda i: (i, 0)),
        pl.BlockSpec((BB, D), lambda i: (i, 0)),
        pl.BlockSpec((1, D), lambda i: (0, 0)),
        pl.BlockSpec((1, D), lambda i: (0, 0)),
        pl.BlockSpec((1, D), lambda i: (0, 0)),
        pl.BlockSpec((1, D), lambda i: (0, 0)),
        pl.BlockSpec((1, 1, BB), lambda i: (i, 0, 0)),
        pl.BlockSpec((D, DOUT), lambda i: (0, 0)),
        pl.BlockSpec((1, DOUT), lambda i: (0, 0)),
    ],
    out_specs=pl.BlockSpec((G, DOUT), lambda i: (0, 0)),
    out_shape=jax.ShapeDtypeStruct((G, DOUT), jnp.float32),
    scratch_shapes=[
        pltpu.VMEM((G, D), jnp.float32),
        pltpu.VMEM((G, 1), jnp.float32),
    ],
)


def kernel(x, edge_index, batch, W_in, b_in, W_conv, b_conv, gamma, beta,
           W_fc, b_fc):
    src = edge_index[0].reshape(NW, EPW)
    dst = edge_index[1].reshape(NW, EPW)
    pad = EPAD - EPW
    src_p = jnp.pad(src, ((0, 0), (0, pad))).reshape(NW, ROWS, CH)
    dst_p = jnp.pad(dst, ((0, 0), (0, pad)),
                    constant_values=PAD_DST).reshape(NW, ROWS, CH)

    degp = _deg_kernel()(dst_p)
    degt = degp[:, :N].T

    h, g = _proj_kernel(x, W_in, b_in.reshape(1, D), W_conv, degt)
    parts = _msg_kernel()(g, src_p, dst_p)
    h2, s1, s2 = _agg_kernel(g, parts, degt, b_conv.reshape(1, D))
    out = _fin_kernel(h2, h, s1, s2, gamma.reshape(1, D), beta.reshape(1, D),
                      batch.reshape(NB, 1, BB), W_fc, b_fc.reshape(1, DOUT))
    return out


# speedup vs baseline: 44.4406x; 2.6218x over previous
import functools

import jax
import jax.numpy as jnp
from jax import lax
from jax.experimental import pallas as pl
from jax.experimental.pallas import tpu as pltpu
from jax.experimental.pallas import tpu_sc as plsc

N = 10000
E = 320000
D = 128
DOUT = 64
G = 16
EPS = 1e-5

NC = 2
NS = 16
NW = NC * NS
EPW = E // NW
CH = 64
ROWS = 160
EPAD = ROWS * CH
NPAD = EPAD
RPT = NPAD // NS

BB = 2000
NB = N // BB

NBUF = 4
SB = 40
NSB = ROWS // SB


def _deg_body(dst_hbm, out_hbm, dst_v, hist):
    c = lax.axis_index("c")
    s = lax.axis_index("s")
    wid = s * NC + c
    pltpu.sync_copy(dst_hbm.at[wid], dst_v)

    def zrow(j, carry):
        hist[pl.ds(j * 16, 16)] = jnp.zeros((16,), jnp.float32)
        return carry

    lax.fori_loop(0, EPAD // 16, zrow, 0)
    ones = jnp.ones((16,), jnp.float32)

    def erow(j, carry):
        for k in range(CH // 16):
            v = dst_v[j, pl.ds(k * 16, 16)]
            plsc.addupdate_scatter(hist, [v], ones)
        return carry

    lax.fori_loop(0, ROWS, erow, 0)
    pltpu.sync_copy(hist, out_hbm.at[wid])


@functools.lru_cache(maxsize=None)
def _deg_kernel():
    mesh = plsc.VectorSubcoreMesh(core_axis_name="c", subcore_axis_name="s")
    return pl.kernel(
        _deg_body,
        out_type=jax.ShapeDtypeStruct((NW, EPAD), jnp.float32),
        mesh=mesh,
        scratch_types=[
            pltpu.VMEM((ROWS, CH), jnp.int32),
            pltpu.VMEM((EPAD,), jnp.float32),
        ],
        compiler_params=pltpu.CompilerParams(needs_layout_passes=False),
    )


def _msg_body(g_hbm, src_hbm, dst_hbm, out_hbm, src_v, dst_v, rows_v, acc_sh,
              gsems, ssems):
    c = lax.axis_index("c")
    s = lax.axis_index("s")
    wid = s * NC + c

    def zrow(j, carry):
        for k in range(D // 16):
            rows_v[0, j, pl.ds(k * 16, 16)] = jnp.zeros((16,), jnp.float32)
        return carry

    lax.fori_loop(0, CH, zrow, 0)
    for k in range(RPT // CH):
        pltpu.sync_copy(rows_v.at[0], acc_sh.at[pl.ds(s * RPT + k * CH, CH)])
    plsc.subcore_barrier()

    for sb in range(NSB):
        pltpu.sync_copy(src_hbm.at[wid, pl.ds(sb * SB, SB)], src_v)
        pltpu.sync_copy(dst_hbm.at[wid, pl.ds(sb * SB, SB)], dst_v)
        g_desc = [None] * NBUF
        s_desc = [None] * NBUF
        for j in range(SB + NBUF - 1):
            if j < SB:
                b = j % NBUF
                if s_desc[b] is not None:
                    s_desc[b].wait()
                    s_desc[b] = None
                g_desc[b] = pltpu.async_copy(
                    g_hbm.at[src_v.at[j]], rows_v.at[b], gsems.at[b])
            jj = j - (NBUF - 1)
            if jj >= 0:
                b2 = jj % NBUF
                g_desc[b2].wait()
                s_desc[b2] = pltpu.async_copy(
                    rows_v.at[b2], acc_sh.at[dst_v.at[jj]], ssems.at[b2],
                    add=True)
        for b in range(NBUF):
            if s_desc[b] is not None:
                s_desc[b].wait()

    plsc.subcore_barrier()
    for k in range(RPT // CH):
        off = s * RPT + k * CH
        pltpu.sync_copy(acc_sh.at[pl.ds(off, CH)], out_hbm.at[c, pl.ds(off, CH)])


@functools.lru_cache(maxsize=None)
def _msg_kernel():
    mesh = plsc.VectorSubcoreMesh(core_axis_name="c", subcore_axis_name="s")
    return pl.kernel(
        _msg_body,
        out_type=jax.ShapeDtypeStruct((NC, NPAD, D), jnp.float32),
        mesh=mesh,
        scratch_types=[
            pltpu.VMEM((SB, CH), jnp.int32),
            pltpu.VMEM((SB, CH), jnp.int32),
            pltpu.VMEM((NBUF, CH, D), jnp.float32),
            pltpu.VMEM_SHARED((NPAD, D), jnp.float32),
            pltpu.SemaphoreType.DMA((NBUF,)),
            pltpu.SemaphoreType.DMA((NBUF,)),
        ],
        compiler_params=pltpu.CompilerParams(needs_layout_passes=False),
    )


def _proj_body(x_ref, win_ref, bin_ref, wconv_ref, degt_ref, h_ref, g_ref):
    h = jnp.dot(x_ref[...], win_ref[...], preferred_element_type=jnp.float32)
    h = h + bin_ref[...]
    hw = jnp.dot(h, wconv_ref[...], preferred_element_type=jnp.float32)
    deg = jnp.sum(degt_ref[...], axis=1, keepdims=True) + 1.0
    dinv = lax.rsqrt(deg)
    h_ref[...] = h
    g_ref[...] = hw * dinv


_proj_kernel = pl.pallas_call(
    _proj_body,
    grid=(NB,),
    in_specs=[
        pl.BlockSpec((BB, D), lambda i: (i, 0)),
        pl.BlockSpec((D, D), lambda i: (0, 0)),
        pl.BlockSpec((1, D), lambda i: (0, 0)),
        pl.BlockSpec((D, D), lambda i: (0, 0)),
        pl.BlockSpec((BB, NW), lambda i: (i, 0)),
    ],
    out_specs=[
        pl.BlockSpec((BB, D), lambda i: (i, 0)),
        pl.BlockSpec((BB, D), lambda i: (i, 0)),
    ],
    out_shape=[
        jax.ShapeDtypeStruct((N, D), jnp.float32),
        jax.ShapeDtypeStruct((N, D), jnp.float32),
    ],
)


def _agg_body(g_ref, p_ref, degt_ref, bconv_ref, h2_ref, s1_ref, s2_ref):
    i = pl.program_id(0)
    deg = jnp.sum(degt_ref[...], axis=1, keepdims=True) + 1.0
    dinv = lax.rsqrt(deg)
    p = p_ref[...]
    h2 = dinv * (g_ref[...] + p[0] + p[1]) + bconv_ref[...]
    h2_ref[...] = h2

    @pl.when(i == 0)
    def _():
        s1_ref[...] = jnp.zeros_like(s1_ref)
        s2_ref[...] = jnp.zeros_like(s2_ref)

    s1_ref[...] += jnp.sum(h2, axis=0, keepdims=True)
    s2_ref[...] += jnp.sum(h2 * h2, axis=0, keepdims=True)


_agg_kernel = pl.pallas_call(
    _agg_body,
    grid=(NB,),
    in_specs=[
        pl.BlockSpec((BB, D), lambda i: (i, 0)),
        pl.BlockSpec((NC, BB, D), lambda i: (0, i, 0)),
        pl.BlockSpec((BB, NW), lambda i: (i, 0)),
        pl.BlockSpec((1, D), lambda i: (0, 0)),
    ],
    out_specs=[
        pl.BlockSpec((BB, D), lambda i: (i, 0)),
        pl.BlockSpec((1, D), lambda i: (0, 0)),
        pl.BlockSpec((1, D), lambda i: (0, 0)),
    ],
    out_shape=[
        jax.ShapeDtypeStruct((N, D), jnp.float32),
        jax.ShapeDtypeStruct((1, D), jnp.float32),
        jax.ShapeDtypeStruct((1, D), jnp.float32),
    ],
)


def _fin_body(h2_ref, h_ref, s1_ref, s2_ref, gamma_ref, beta_ref, batch_ref,
              wfc_ref, bfc_ref, out_ref, pool_acc, cnt_acc):
    i = pl.program_id(0)
    mean = s1_ref[...] * (1.0 / N)
    var = s2_ref[...] * (1.0 / N) - mean * mean
    inv = lax.rsqrt(var + EPS)
    hn = (h2_ref[...] - mean) * inv * gamma_ref[...] + beta_ref[...]
    hf = jnp.maximum(hn, 0.0) + h_ref[...]
    b = batch_ref[...][0]
    seg = lax.broadcasted_iota(jnp.int32, (G, BB), 0)
    mask = (seg == b).astype(jnp.float32)

    @pl.when(i == 0)
    def _():
        pool_acc[...] = jnp.zeros_like(pool_acc)
        cnt_acc[...] = jnp.zeros_like(cnt_acc)

    pool_acc[...] += jnp.dot(mask, hf, preferred_element_type=jnp.float32)
    cnt_acc[...] += jnp.sum(mask, axis=1, keepdims=True)

    @pl.when(i == pl.num_programs(0) - 1)
    def _():
        pooled = pool_acc[...] / jnp.maximum(cnt_acc[...], 1.0)
        z = jnp.dot(pooled, wfc_ref[...], preferred_element_type=jnp.float32)
        z = z + bfc_ref[...]
        out_ref[...] = 1.0 / (1.0 + jnp.exp(-z))


_fin_kernel = pl.pallas_call(
    _fin_body,
    grid=(NB,),
    in_specs=[
        pl.BlockSpec((BB, D), lambda i: (i, 0)),
        pl.BlockSpec((BB, D), lambda i: (i, 0)),
        pl.BlockSpec((1, D), lambda i: (0, 0)),
        pl.BlockSpec((1, D), lambda i: (0, 0)),
        pl.BlockSpec((1, D), lambda i: (0, 0)),
        pl.BlockSpec((1, D), lambda i: (0, 0)),
        pl.BlockSpec((1, 1, BB), lambda i: (i, 0, 0)),
        pl.BlockSpec((D, DOUT), lambda i: (0, 0)),
        pl.BlockSpec((1, DOUT), lambda i: (0, 0)),
    ],
    out_specs=pl.BlockSpec((G, DOUT), lambda i: (0, 0)),
    out_shape=jax.ShapeDtypeStruct((G, DOUT), jnp.float32),
    scratch_shapes=[
        pltpu.VMEM((G, D), jnp.float32),
        pltpu.VMEM((G, 1), jnp.float32),
    ],
)


def kernel(x, edge_index, batch, W_in, b_in, W_conv, b_conv, gamma, beta,
           W_fc, b_fc):
    src = edge_index[0].reshape(NW, EPW)
    dst = edge_index[1].reshape(NW, EPW)
    pad = EPAD - EPW
    wids = jnp.arange(NW, dtype=jnp.int32)[:, None]
    ks = jnp.arange(pad, dtype=jnp.int32)[None, :]
    src_fill = (wids * 311 + ks) % N
    dst_fill = N + (wids * 7 + ks) % (NPAD - N)
    src_p = jnp.concatenate([src, src_fill], axis=1).reshape(NW, ROWS, CH)
    dst_p = jnp.concatenate([dst, dst_fill], axis=1).reshape(NW, ROWS, CH)

    degp = _deg_kernel()(dst_p)
    degt = degp[:, :N].T

    h, g = _proj_kernel(x, W_in, b_in.reshape(1, D), W_conv, degt)
    parts = _msg_kernel()(g, src_p, dst_p)
    h2, s1, s2 = _agg_kernel(g, parts, degt, b_conv.reshape(1, D))
    out = _fin_kernel(h2, h, s1, s2, gamma.reshape(1, D), beta.reshape(1, D),
                      batch.reshape(NB, 1, BB), W_fc, b_fc.reshape(1, DOUT))
    return out


# speedup vs baseline: 45.5423x; 1.0248x over previous
import functools

import jax
import jax.numpy as jnp
from jax import lax
from jax.experimental import pallas as pl
from jax.experimental.pallas import tpu as pltpu
from jax.experimental.pallas import tpu_sc as plsc

N = 10000
E = 320000
D = 128
DOUT = 64
G = 16
EPS = 1e-5

NC = 2
NS = 16
NW = NC * NS
EPW = E // NW
CH = 64
ROWS = 160
EPAD = ROWS * CH
NPAD = EPAD
RPT = NPAD // NS

BB = 2000
NB = N // BB

NBUF = 4
SB = 40
NSB = ROWS // SB


def _deg_body(dst_hbm, out_hbm, dst_v, hist):
    c = lax.axis_index("c")
    s = lax.axis_index("s")
    wid = s * NC + c
    pltpu.sync_copy(dst_hbm.at[wid], dst_v)

    def zrow(j, carry):
        hist[pl.ds(j * 16, 16)] = jnp.zeros((16,), jnp.float32)
        return carry

    lax.fori_loop(0, EPAD // 16, zrow, 0)
    ones = jnp.ones((16,), jnp.float32)

    def erow(j, carry):
        for k in range(CH // 16):
            v = dst_v[j, pl.ds(k * 16, 16)]
            plsc.addupdate_scatter(hist, [v], ones)
        return carry

    lax.fori_loop(0, ROWS, erow, 0)
    pltpu.sync_copy(hist, out_hbm.at[wid])


@functools.lru_cache(maxsize=None)
def _deg_kernel():
    mesh = plsc.VectorSubcoreMesh(core_axis_name="c", subcore_axis_name="s")
    return pl.kernel(
        _deg_body,
        out_type=jax.ShapeDtypeStruct((NW, EPAD), jnp.float32),
        mesh=mesh,
        scratch_types=[
            pltpu.VMEM((ROWS, CH), jnp.int32),
            pltpu.VMEM((EPAD,), jnp.float32),
        ],
        compiler_params=pltpu.CompilerParams(needs_layout_passes=False),
    )


def _msg_body(g_hbm, src_hbm, dst_hbm, out_hbm, src_v, dst_v, rows_v, acc_sh,
              gsems, ssems):
    c = lax.axis_index("c")
    s = lax.axis_index("s")
    wid = s * NC + c

    def zrow(j, carry):
        for k in range(D // 16):
            rows_v[0, j, pl.ds(k * 16, 16)] = jnp.zeros((16,), jnp.float32)
        return carry

    lax.fori_loop(0, CH, zrow, 0)
    for k in range(RPT // CH):
        pltpu.sync_copy(rows_v.at[0], acc_sh.at[pl.ds(s * RPT + k * CH, CH)])
    plsc.subcore_barrier()

    for sb in range(NSB):
        pltpu.sync_copy(src_hbm.at[wid, pl.ds(sb * SB, SB)], src_v)
        pltpu.sync_copy(dst_hbm.at[wid, pl.ds(sb * SB, SB)], dst_v)
        g_desc = [None] * NBUF
        s_desc = [None] * NBUF
        for j in range(SB + NBUF - 1):
            if j < SB:
                b = j % NBUF
                if s_desc[b] is not None:
                    s_desc[b].wait()
                    s_desc[b] = None
                g_desc[b] = pltpu.async_copy(
                    g_hbm.at[src_v.at[j]], rows_v.at[b], gsems.at[b])
            jj = j - (NBUF - 1)
            if jj >= 0:
                b2 = jj % NBUF
                g_desc[b2].wait()
                s_desc[b2] = pltpu.async_copy(
                    rows_v.at[b2], acc_sh.at[dst_v.at[jj]], ssems.at[b2],
                    add=True)
        for b in range(NBUF):
            if s_desc[b] is not None:
                s_desc[b].wait()

    plsc.subcore_barrier()
    for k in range(RPT // CH):
        off = s * RPT + k * CH
        pltpu.sync_copy(acc_sh.at[pl.ds(off, CH)], out_hbm.at[c, pl.ds(off, CH)])


@functools.lru_cache(maxsize=None)
def _msg_kernel():
    mesh = plsc.VectorSubcoreMesh(core_axis_name="c", subcore_axis_name="s")
    return pl.kernel(
        _msg_body,
        out_type=jax.ShapeDtypeStruct((NC, NPAD, D), jnp.float32),
        mesh=mesh,
        scratch_types=[
            pltpu.VMEM((SB, CH), jnp.int32),
            pltpu.VMEM((SB, CH), jnp.int32),
            pltpu.VMEM((NBUF, CH, D), jnp.float32),
            pltpu.VMEM_SHARED((NPAD, D), jnp.float32),
            pltpu.SemaphoreType.DMA((NBUF,)),
            pltpu.SemaphoreType.DMA((NBUF,)),
        ],
        compiler_params=pltpu.CompilerParams(needs_layout_passes=False),
    )


def _proj_body(x_ref, win_ref, bin_ref, wconv_ref, degt_ref, h_ref, g_ref):
    h = jnp.dot(x_ref[...], win_ref[...], preferred_element_type=jnp.float32)
    h = h + bin_ref[...]
    hw = jnp.dot(h, wconv_ref[...], preferred_element_type=jnp.float32)
    deg = jnp.sum(degt_ref[...], axis=1, keepdims=True) + 1.0
    dinv = lax.rsqrt(deg)
    h_ref[...] = h
    g_ref[...] = hw * dinv


_proj_kernel = pl.pallas_call(
    _proj_body,
    grid=(NB,),
    in_specs=[
        pl.BlockSpec((BB, D), lambda i: (i, 0)),
        pl.BlockSpec((D, D), lambda i: (0, 0)),
        pl.BlockSpec((1, D), lambda i: (0, 0)),
        pl.BlockSpec((D, D), lambda i: (0, 0)),
        pl.BlockSpec((BB, NW), lambda i: (i, 0)),
    ],
    out_specs=[
        pl.BlockSpec((BB, D), lambda i: (i, 0)),
        pl.BlockSpec((BB, D), lambda i: (i, 0)),
    ],
    out_shape=[
        jax.ShapeDtypeStruct((N, D), jnp.float32),
        jax.ShapeDtypeStruct((N, D), jnp.float32),
    ],
)


def _d_body(g_ref, p_ref, degt_ref, bconv_ref, h_ref, batch_ref, gamma_ref,
            beta_ref, wfc_ref, bfc_ref, out_ref, h2_buf, s1, s2, pool_acc,
            cnt_acc):
    i = pl.program_id(0)

    @pl.when(i == 0)
    def _():
        s1[...] = jnp.zeros_like(s1)
        s2[...] = jnp.zeros_like(s2)
        pool_acc[...] = jnp.zeros_like(pool_acc)
        cnt_acc[...] = jnp.zeros_like(cnt_acc)

    @pl.when(i < NB)
    def _():
        deg = jnp.sum(degt_ref[...], axis=1, keepdims=True) + 1.0
        dinv = lax.rsqrt(deg)
        p = p_ref[...]
        h2 = dinv * (g_ref[...] + p[0] + p[1]) + bconv_ref[...]
        h2_buf[pl.ds(i * BB, BB), :] = h2
        s1[...] += jnp.sum(h2, axis=0, keepdims=True)
        s2[...] += jnp.sum(h2 * h2, axis=0, keepdims=True)

    @pl.when(i >= NB)
    def _():
        j = i - NB
        mean = s1[...] * (1.0 / N)
        var = s2[...] * (1.0 / N) - mean * mean
        inv = lax.rsqrt(var + EPS)
        h2 = h2_buf[pl.ds(j * BB, BB), :]
        hn = (h2 - mean) * inv * gamma_ref[...] + beta_ref[...]
        hf = jnp.maximum(hn, 0.0) + h_ref[...]
        b = batch_ref[...][0]
        seg = lax.broadcasted_iota(jnp.int32, (G, BB), 0)
        mask = (seg == b).astype(jnp.float32)
        pool_acc[...] += jnp.dot(mask, hf, preferred_element_type=jnp.float32)
        cnt_acc[...] += jnp.sum(mask, axis=1, keepdims=True)

    @pl.when(i == 2 * NB - 1)
    def _():
        pooled = pool_acc[...] / jnp.maximum(cnt_acc[...], 1.0)
        z = jnp.dot(pooled, wfc_ref[...], preferred_element_type=jnp.float32)
        z = z + bfc_ref[...]
        out_ref[...] = 1.0 / (1.0 + jnp.exp(-z))


def _clamp1(i):
    return jnp.minimum(i, NB - 1)


def _shift(i):
    return jnp.maximum(i, NB) - NB


_d_kernel = pl.pallas_call(
    _d_body,
    grid=(2 * NB,),
    in_specs=[
        pl.BlockSpec((BB, D), lambda i: (_clamp1(i), 0)),
        pl.BlockSpec((NC, BB, D), lambda i: (0, _clamp1(i), 0)),
        pl.BlockSpec((BB, NW), lambda i: (_clamp1(i), 0)),
        pl.BlockSpec((1, D), lambda i: (0, 0)),
        pl.BlockSpec((BB, D), lambda i: (_shift(i), 0)),
        pl.BlockSpec((1, 1, BB), lambda i: (_shift(i), 0, 0)),
        pl.BlockSpec((1, D), lambda i: (0, 0)),
        pl.BlockSpec((1, D), lambda i: (0, 0)),
        pl.BlockSpec((D, DOUT), lambda i: (0, 0)),
        pl.BlockSpec((1, DOUT), lambda i: (0, 0)),
    ],
    out_specs=pl.BlockSpec((G, DOUT), lambda i: (0, 0)),
    out_shape=jax.ShapeDtypeStruct((G, DOUT), jnp.float32),
    scratch_shapes=[
        pltpu.VMEM((N, D), jnp.float32),
        pltpu.VMEM((1, D), jnp.float32),
        pltpu.VMEM((1, D), jnp.float32),
        pltpu.VMEM((G, D), jnp.float32),
        pltpu.VMEM((G, 1), jnp.float32),
    ],
)


def kernel(x, edge_index, batch, W_in, b_in, W_conv, b_conv, gamma, beta,
           W_fc, b_fc):
    src = edge_index[0].reshape(NW, EPW)
    dst = edge_index[1].reshape(NW, EPW)
    pad = EPAD - EPW
    wids = jnp.arange(NW, dtype=jnp.int32)[:, None]
    ks = jnp.arange(pad, dtype=jnp.int32)[None, :]
    src_fill = (wids * 311 + ks) % N
    dst_fill = N + (wids * 7 + ks) % (NPAD - N)
    src_p = jnp.concatenate([src, src_fill], axis=1).reshape(NW, ROWS, CH)
    dst_p = jnp.concatenate([dst, dst_fill], axis=1).reshape(NW, ROWS, CH)

    degp = _deg_kernel()(dst_p)
    degt = degp[:, :N].T

    h, g = _proj_kernel(x, W_in, b_in.reshape(1, D), W_conv, degt)
    parts = _msg_kernel()(g, src_p, dst_p)
    out = _d_kernel(g, parts, degt, b_conv.reshape(1, D), h,
                    batch.reshape(NB, 1, BB), gamma.reshape(1, D),
                    beta.reshape(1, D), W_fc, b_fc.reshape(1, DOUT))
    return out
